# baseline (speedup 1.0000x reference)
import functools

import jax
import jax.numpy as jnp
import numpy as np
from jax import lax
from jax.experimental import pallas as pl
from jax.experimental.pallas import tpu as pltpu
from jax.experimental.pallas import tpu_sc as plsc

VOCAB = 1_000_000
D = 32
DW = D // 2
BATCH = 16384
HIST = 200
B = BATCH * HIST

NC = 2
NS = 16
NW = NC * NS
BPW = B // NW

C = 1280
K = 3
NCHUNK = BPW // C
ROW_UNROLL = 4

SCALE = np.float32(np.sqrt(np.float32(D)))

_mesh = plsc.VectorSubcoreMesh(
    core_axis_name="c", subcore_axis_name="s", num_cores=NC, num_subcores=NS
)


def _rne16(x):
    u = jax.lax.bitcast_convert_type(x, jnp.uint32)
    return (u + jnp.uint32(0x7FFF) + ((u >> 16) & jnp.uint32(1))) >> 16


_TR_TOTAL = VOCAB * D // 128
_R1 = 1000

def _pack_body(t_ref, o_ref):
    val = t_ref[...].reshape(_R1, 2, 128)
    halves = []
    for e in (val[:, 0, :], val[:, 1, :]):
        for j in range(4):
            lo = _rne16(e[:, 32 * j:32 * j + 16])
            hi = _rne16(e[:, 32 * j + 16:32 * j + 32])
            halves.append(lo | (hi << 16))
    o_ref[...] = jax.lax.bitcast_convert_type(
        jnp.concatenate(halves, axis=1), jnp.int32
    )

_pack_tab = pl.pallas_call(
    _pack_body,
    grid=(_TR_TOTAL // (2 * _R1),),
    in_specs=[pl.BlockSpec((2 * _R1, 128), lambda i: (i, 0))],
    out_specs=pl.BlockSpec((_R1, 128), lambda i: (i, 0)),
    out_shape=jax.ShapeDtypeStruct((_TR_TOTAL // 2, 128), jnp.int32),
)


_PK_TOTAL = B * DW // 128
_R2 = 3200

def _unpack_body(p_ref, o_ref):
    w = jax.lax.bitcast_convert_type(p_ref[...], jnp.uint32)
    lo = jax.lax.bitcast_convert_type(w << 16, jnp.float32) * SCALE
    hi = jax.lax.bitcast_convert_type(w & jnp.uint32(0xFFFF0000),
                                      jnp.float32) * SCALE
    parts = []
    for half in range(2):
        cols = []
        for j in range(4):
            sl = slice(64 * half + 16 * j, 64 * half + 16 * j + 16)
            cols.append(lo[:, sl])
            cols.append(hi[:, sl])
        parts.append(jnp.concatenate(cols, axis=1))
    out = jnp.stack(parts, axis=1)
    o_ref[...] = out.reshape(2 * _R2, 128)

_unpack_out = pl.pallas_call(
    _unpack_body,
    grid=(_PK_TOTAL // _R2,),
    in_specs=[pl.BlockSpec((_R2, 128), lambda i: (i, 0))],
    out_specs=pl.BlockSpec((2 * _R2, 128), lambda i: (i, 0)),
    out_shape=jax.ShapeDtypeStruct((2 * _PK_TOTAL, 128), jnp.float32),
)


@functools.partial(
    pl.kernel,
    out_type=jax.ShapeDtypeStruct((B, DW), jnp.int32),
    mesh=_mesh,
    compiler_params=pltpu.CompilerParams(use_tc_tiling_on_sc=False),
    scratch_types=[
        pltpu.VMEM((K, 1, C), jnp.int32),
        pltpu.VMEM((K, C, DW), jnp.int32),
        pltpu.VMEM((K, C, DW), jnp.int32),
        [pltpu.SemaphoreType.DMA] * K,
        [pltpu.SemaphoreType.DMA] * K,
        [pltpu.SemaphoreType.DMA] * K,
    ],
)
def _embed_kernel(x_hbm, tab_hbm, out_hbm, idx_v, rows_v, srows_v,
                  isems, gsems, ssems):
    wid = lax.axis_index("s") * NC + lax.axis_index("c")
    base = wid * BPW
    xrow0 = wid * NCHUNK

    def prefetch_idx(b, c):
        return pltpu.async_copy(
            x_hbm.at[pl.ds(xrow0 + c, 1)], idx_v.at[b], isems[b]
        )

    def fire_gather(b, idx_copy):
        idx_copy.wait()
        pltpu.async_copy(tab_hbm.at[idx_v.at[b, 0]], rows_v.at[b], gsems[b])

    def wait_gather(b):
        pltpu.make_async_copy(
            tab_hbm.at[idx_v.at[b, 0]], rows_v.at[b], gsems[b]
        ).wait()

    def wait_store(b, c):
        pltpu.make_async_copy(
            srows_v.at[b], out_hbm.at[pl.ds(base + c * C, C)], ssems[b]
        ).wait()

    def copy_chunk(b):
        @pl.loop(0, C, step=ROW_UNROLL)
        def _(i):
            for di in range(ROW_UNROLL):
                for h in range(DW // 16):
                    srows_v[b, i + di, pl.ds(16 * h, 16)] = (
                        rows_v[b, i + di, pl.ds(16 * h, 16)]
                    )

    def start_store(b, c):
        pltpu.async_copy(
            srows_v.at[b], out_hbm.at[pl.ds(base + c * C, C)], ssems[b]
        )

    def turn(b, c, *, skip_store_wait, fire):
        wait_gather(b)
        if fire:
            icopy = prefetch_idx(b, c + K)
        if not skip_store_wait:
            wait_store(b, c - K)
        copy_chunk(b)
        if fire:
            fire_gather(b, icopy)
        start_store(b, c)

    for c in range(K):
        fire_gather(c, prefetch_idx(c, c))

    for c in range(K):
        turn(c, c, skip_store_wait=True, fire=True)

    full_lo, full_hi = K, NCHUNK - K
    n_loop = ((full_hi - full_lo) // K) * K

    @pl.loop(full_lo, full_lo + n_loop, step=K)
    def _(g):
        for db in range(K):
            turn(db, g + db, skip_store_wait=False, fire=True)

    for c in range(full_lo + n_loop, full_hi):
        turn(c % K, c, skip_store_wait=False, fire=True)

    for c in range(full_hi, NCHUNK):
        turn(c % K, c, skip_store_wait=False, fire=False)

    for c in range(NCHUNK - K, NCHUNK):
        wait_store(c % K, c)


def kernel(x, input_embedding_table):
    packed_tab = _pack_tab(input_embedding_table.reshape(_TR_TOTAL, 128))
    tab_sc = packed_tab.reshape(VOCAB, DW)
    x2d = x.reshape(B // C, C)
    packed = _embed_kernel(x2d, tab_sc)
    return packed


# speedup vs baseline: 1.0190x; 1.0190x over previous
import functools

import jax
import jax.numpy as jnp
import numpy as np
from jax import lax
from jax.experimental import pallas as pl
from jax.experimental.pallas import tpu as pltpu
from jax.experimental.pallas import tpu_sc as plsc

VOCAB = 1_000_000
D = 32
BATCH = 16384
HIST = 200
B = BATCH * HIST

NC = 2
NS = 16
NW = NC * NS
BPW = B // NW

C = 400
K = 4
NCHUNK = BPW // C
ROW_UNROLL = 4

SCALE = np.float32(np.sqrt(np.float32(D)))

_mesh = plsc.VectorSubcoreMesh(
    core_axis_name="c", subcore_axis_name="s", num_cores=NC, num_subcores=NS
)


@functools.partial(
    pl.kernel,
    out_type=jax.ShapeDtypeStruct((B, D), jnp.float32),
    mesh=_mesh,
    compiler_params=pltpu.CompilerParams(use_tc_tiling_on_sc=False),
    scratch_types=[
        pltpu.VMEM((K, 1, C), jnp.int32),
        pltpu.VMEM((K, C, D), jnp.float32),
        pltpu.VMEM((K, C, D), jnp.float32),
        [pltpu.SemaphoreType.DMA] * K,
        [pltpu.SemaphoreType.DMA] * K,
        [pltpu.SemaphoreType.DMA] * K,
    ],
)
def _embed_kernel(x_hbm, tab_hbm, out_hbm, idx_v, rows_v, srows_v,
                  isems, gsems, ssems):
    wid = lax.axis_index("s") * NC + lax.axis_index("c")
    base = wid * BPW
    xrow0 = wid * NCHUNK

    def prefetch_idx(b, c):
        return pltpu.async_copy(
            x_hbm.at[pl.ds(xrow0 + c, 1)], idx_v.at[b], isems[b]
        )

    def fire_gather(b, idx_copy):
        idx_copy.wait()
        pltpu.async_copy(tab_hbm.at[idx_v.at[b, 0]], rows_v.at[b], gsems[b])

    def wait_gather(b):
        pltpu.make_async_copy(
            tab_hbm.at[idx_v.at[b, 0]], rows_v.at[b], gsems[b]
        ).wait()

    def wait_store(b, c):
        pltpu.make_async_copy(
            srows_v.at[b], out_hbm.at[pl.ds(base + c * C, C)], ssems[b]
        ).wait()

    def scale_chunk(b):
        @pl.loop(0, C, step=ROW_UNROLL)
        def _(i):
            for di in range(ROW_UNROLL):
                for h in range(D // 16):
                    v = rows_v[b, i + di, pl.ds(16 * h, 16)]
                    srows_v[b, i + di, pl.ds(16 * h, 16)] = v * SCALE

    def start_store(b, c):
        pltpu.async_copy(
            srows_v.at[b], out_hbm.at[pl.ds(base + c * C, C)], ssems[b]
        )

    def turn(b, c, *, skip_store_wait, fire):
        wait_gather(b)
        if fire:
            icopy = prefetch_idx(b, c + K)
        if not skip_store_wait:
            wait_store(b, c - K)
        scale_chunk(b)
        if fire:
            fire_gather(b, icopy)
        start_store(b, c)

    for c in range(K):
        fire_gather(c, prefetch_idx(c, c))

    for c in range(K):
        turn(c, c, skip_store_wait=True, fire=True)

    full_lo, full_hi = K, NCHUNK - K
    n_loop = ((full_hi - full_lo) // K) * K

    @pl.loop(full_lo, full_lo + n_loop, step=K)
    def _(g):
        for db in range(K):
            turn(db, g + db, skip_store_wait=False, fire=True)

    for c in range(full_lo + n_loop, full_hi):
        turn(c % K, c, skip_store_wait=False, fire=True)

    for c in range(full_hi, NCHUNK):
        turn(c % K, c, skip_store_wait=False, fire=False)

    for c in range(NCHUNK - K, NCHUNK):
        wait_store(c % K, c)


def kernel(x, input_embedding_table):
    x2d = x.reshape(B // C, C)
    out = _embed_kernel(x2d, input_embedding_table)
    return out.reshape(BATCH, HIST, D)


# speedup vs baseline: 1.0212x; 1.0022x over previous
import functools

import jax
import jax.numpy as jnp
import numpy as np
from jax import lax
from jax.experimental import pallas as pl
from jax.experimental.pallas import tpu as pltpu
from jax.experimental.pallas import tpu_sc as plsc

VOCAB = 1_000_000
D = 32
BATCH = 16384
HIST = 200
B = BATCH * HIST

NC = 2
NS = 16
NW = NC * NS
BPW = B // NW

C = 640
K = 3
NCHUNK = BPW // C
ROW_UNROLL = 4

SCALE = np.float32(np.sqrt(np.float32(D)))

_mesh = plsc.VectorSubcoreMesh(
    core_axis_name="c", subcore_axis_name="s", num_cores=NC, num_subcores=NS
)


@functools.partial(
    pl.kernel,
    out_type=jax.ShapeDtypeStruct((B, D), jnp.float32),
    mesh=_mesh,
    compiler_params=pltpu.CompilerParams(use_tc_tiling_on_sc=False),
    scratch_types=[
        pltpu.VMEM((K, 1, C), jnp.int32),
        pltpu.VMEM((K, C, D), jnp.float32),
        pltpu.VMEM((K, C, D), jnp.float32),
        [pltpu.SemaphoreType.DMA] * K,
        [pltpu.SemaphoreType.DMA] * K,
        [pltpu.SemaphoreType.DMA] * K,
    ],
)
def _embed_kernel(x_hbm, tab_hbm, out_hbm, idx_v, rows_v, srows_v,
                  isems, gsems, ssems):
    wid = lax.axis_index("s") * NC + lax.axis_index("c")
    base = wid * BPW
    xrow0 = wid * NCHUNK

    def prefetch_idx(b, c):
        return pltpu.async_copy(
            x_hbm.at[pl.ds(xrow0 + c, 1)], idx_v.at[b], isems[b]
        )

    def fire_gather(b, idx_copy):
        idx_copy.wait()
        pltpu.async_copy(tab_hbm.at[idx_v.at[b, 0]], rows_v.at[b], gsems[b])

    def wait_gather(b):
        pltpu.make_async_copy(
            tab_hbm.at[idx_v.at[b, 0]], rows_v.at[b], gsems[b]
        ).wait()

    def wait_store(b, c):
        pltpu.make_async_copy(
            srows_v.at[b], out_hbm.at[pl.ds(base + c * C, C)], ssems[b]
        ).wait()

    def scale_chunk(b):
        @pl.loop(0, C, step=ROW_UNROLL)
        def _(i):
            for di in range(ROW_UNROLL):
                for h in range(D // 16):
                    v = rows_v[b, i + di, pl.ds(16 * h, 16)]
                    srows_v[b, i + di, pl.ds(16 * h, 16)] = v * SCALE

    def start_store(b, c):
        pltpu.async_copy(
            srows_v.at[b], out_hbm.at[pl.ds(base + c * C, C)], ssems[b]
        )

    def turn(b, c, *, skip_store_wait, fire):
        wait_gather(b)
        if fire:
            icopy = prefetch_idx(b, c + K)
        if not skip_store_wait:
            wait_store(b, c - K)
        scale_chunk(b)
        if fire:
            fire_gather(b, icopy)
        start_store(b, c)

    for c in range(K):
        fire_gather(c, prefetch_idx(c, c))

    for c in range(K):
        turn(c, c, skip_store_wait=True, fire=True)

    full_lo, full_hi = K, NCHUNK - K
    n_loop = ((full_hi - full_lo) // K) * K

    @pl.loop(full_lo, full_lo + n_loop, step=K)
    def _(g):
        for db in range(K):
            turn(db, g + db, skip_store_wait=False, fire=True)

    for c in range(full_lo + n_loop, full_hi):
        turn(c % K, c, skip_store_wait=False, fire=True)

    for c in range(full_hi, NCHUNK):
        turn(c % K, c, skip_store_wait=False, fire=False)

    for c in range(NCHUNK - K, NCHUNK):
        wait_store(c % K, c)


def kernel(x, input_embedding_table):
    x2d = x.reshape(B // C, C)
    out = _embed_kernel(x2d, input_embedding_table)
    return out.reshape(BATCH, HIST, D)
